# baseline (speedup 1.0000x reference)
import functools

import jax
import jax.numpy as jnp
from jax import lax
from jax.experimental import pallas as pl
from jax.experimental.pallas import tpu as pltpu
from jax.experimental.pallas import tpu_sc as plsc

N = 20000
H = 256
W = 256
GP = 20480
GCH = GP // 16
TS = 16
TGX = W // TS
TGY = H // TS
T = TGX * TGY
K = 512
PW = 16
TWO_T = 24.0
RCH = 64
NKC = K // RCH
NSUB = 32
TPS = T // NSUB
PBLK = 2048


def _proj_kernel(xyz_ref, chol_ref, fdc_ref, op_ref, out_ref):
    i = pl.program_id(0)
    mx = jnp.tanh(xyz_ref[0:1, :])
    my = jnp.tanh(xyz_ref[1:2, :])
    x = 0.5 * (mx + 1.0) * float(W)
    y = 0.5 * (my + 1.0) * float(H)
    l1 = chol_ref[0:1, :] + 0.5
    l2 = chol_ref[1:2, :]
    l3 = chol_ref[2:3, :] + 0.5
    a = l1 * l1
    b = l1 * l2
    c = l2 * l2 + l3 * l3
    inv_det = 1.0 / (a * c - b * b)
    e = 0.5 * c * inv_det
    f = -b * inv_det
    g = 0.5 * a * inv_det
    valid = (i * PBLK + lax.broadcasted_iota(jnp.int32, (1, PBLK), 1)) < N
    r2 = jnp.where(valid, TWO_T * (a + c), -1.0)
    cols = fdc_ref[...] * op_ref[0:1, :]
    zero = jnp.zeros((1, PBLK), jnp.float32)
    out_ref[...] = jnp.concatenate(
        [x, y, e, f, g, cols[0:1], cols[1:2], cols[2:3], r2,
         zero, zero, zero, zero, zero, zero, zero], axis=0)


def _project(xyz_t, chol_t, fdc_t, op_t):
    return pl.pallas_call(
        _proj_kernel,
        grid=(GP // PBLK,),
        in_specs=[
            pl.BlockSpec((2, PBLK), lambda i: (0, i)),
            pl.BlockSpec((3, PBLK), lambda i: (0, i)),
            pl.BlockSpec((3, PBLK), lambda i: (0, i)),
            pl.BlockSpec((1, PBLK), lambda i: (0, i)),
        ],
        out_specs=pl.BlockSpec((PW, PBLK), lambda i: (0, i)),
        out_shape=jax.ShapeDtypeStruct((PW, GP), jnp.float32),
    )(xyz_t, chol_t, fdc_t, op_t)


@functools.cache
def _make_bin_kernel():
    mesh = plsc.VectorSubcoreMesh(core_axis_name="c", subcore_axis_name="s")
    return functools.partial(
        pl.kernel,
        mesh=mesh,
        compiler_params=pltpu.CompilerParams(
            use_tc_tiling_on_sc=False, needs_layout_passes=False),
        out_type=[
            jax.ShapeDtypeStruct((T, K, PW), jnp.float32),
            jax.ShapeDtypeStruct((T,), jnp.int32),
        ],
        scratch_types=[
            pltpu.VMEM((GP,), jnp.float32),
            pltpu.VMEM((GP,), jnp.float32),
            pltpu.VMEM((GP,), jnp.float32),
            pltpu.VMEM((2048,), jnp.int32),
            pltpu.VMEM((K, PW), jnp.float32),
            pltpu.VMEM((16,), jnp.int32),
            pltpu.SemaphoreType.DMA,
        ],
    )(_bin_kernel)


def _bin_kernel(pt_hbm, tbl_hbm, tp_hbm, cnt_hbm, xs, ys, r2s, ids, rows,
                cvec, sem):
    c_ = lax.axis_index("c")
    s_ = lax.axis_index("s")
    wid = s_ * 2 + c_
    pltpu.sync_copy(pt_hbm.at[0], xs)
    pltpu.sync_copy(pt_hbm.at[1], ys)
    pltpu.sync_copy(pt_hbm.at[8], r2s)

    cvals = jnp.zeros((16,), jnp.int32)
    for k in range(TPS):
        tile = wid * TPS + k
        ty = tile // TGX
        tx = tile % TGX
        x0 = tx.astype(jnp.float32) * float(TS) + 0.5
        x1 = x0 + float(TS - 1)
        y0 = ty.astype(jnp.float32) * float(TS) + 0.5
        y1 = y0 + float(TS - 1)

        def init_body(j, _):
            ids[pl.ds(j * 16, 16)] = jnp.full((16,), GP, jnp.int32)
            return 0

        lax.fori_loop(0, K // 16, init_body, 0)

        def scan_body(ci, cnt):
            xv = xs[pl.ds(ci * 16, 16)]
            yv = ys[pl.ds(ci * 16, 16)]
            rv = r2s[pl.ds(ci * 16, 16)]
            dx = jnp.maximum(jnp.maximum(x0 - xv, xv - x1), 0.0)
            dy = jnp.maximum(jnp.maximum(y0 - yv, yv - y1), 0.0)
            m = (dx * dx + dy * dy) <= rv
            iv = ci * 16 + lax.iota(jnp.int32, 16)
            mi = m.astype(jnp.int32)
            csum = plsc.cumsum(mi)
            pos = cnt + csum - 1
            plsc.store_scatter(ids, [pos], iv, mask=m)
            return cnt + jnp.sum(mi)

        cnt = lax.fori_loop(0, GCH, scan_body, 0)
        cvals = jnp.where(lax.iota(jnp.int32, 16) == k, cnt, cvals)

        for j in range(K // 128):
            pltpu.async_copy(
                tbl_hbm.at[ids.at[pl.ds(j * 128, 128)]],
                rows.at[pl.ds(j * 128, 128)], sem).wait()
        pltpu.sync_copy(rows, tp_hbm.at[tile])

    cvec[...] = cvals
    pltpu.sync_copy(cvec.at[pl.ds(0, TPS)],
                    cnt_hbm.at[pl.ds(wid * TPS, TPS)])


def _raster_kernel(cnt_ref, tp_ref, out_ref):
    t = pl.program_id(0)
    kc = pl.program_id(1)
    ty = t // TGX
    tx = t % TGX

    @pl.when(kc == 0)
    def _():
        out_ref[0] = jnp.zeros((3, TS * TS), jnp.float32)

    @pl.when(kc * RCH < cnt_ref[t])
    def _():
        p = tp_ref[0]
        x = p[:, 0:1]
        y = p[:, 1:2]
        e = p[:, 2:3]
        f = p[:, 3:4]
        g = p[:, 4:5]
        li = lax.broadcasted_iota(jnp.int32, (RCH, TS * TS), 1)
        pxv = (tx * TS + (li & (TS - 1))).astype(jnp.float32) + 0.5
        pyv = (ty * TS + (li >> 4)).astype(jnp.float32) + 0.5
        dx = pxv - x
        dy = pyv - y
        sig = dx * (e * dx + f * dy) + g * dy * dy
        alpha = jnp.exp(-sig)
        colsT = p[:, 5:8].T
        acc = lax.dot_general(colsT, alpha, (((1,), (0,)), ((), ())),
                              preferred_element_type=jnp.float32)
        out_ref[0] = out_ref[0] + acc

    @pl.when(kc == NKC - 1)
    def _():
        out_ref[0] = jnp.clip(out_ref[0], 0.0, 1.0)


def _tp_index_map(t, kc, cnt_ref):
    last = jnp.maximum((cnt_ref[t] + (RCH - 1)) // RCH - 1, 0)
    return (t, jnp.minimum(kc, last), 0)


def _raster(tp, cnts):
    return pl.pallas_call(
        _raster_kernel,
        grid_spec=pltpu.PrefetchScalarGridSpec(
            num_scalar_prefetch=1,
            grid=(T, NKC),
            in_specs=[pl.BlockSpec((1, RCH, PW), _tp_index_map)],
            out_specs=pl.BlockSpec((1, 3, TS * TS),
                                   lambda t, kc, cnt_ref: (t, 0, 0)),
        ),
        out_shape=jax.ShapeDtypeStruct((T, 3, TS * TS), jnp.float32),
    )(cnts, tp)


def kernel(_xyz, _cholesky, _opacity, _features_dc, background):
    pad = GP - N
    xyz_t = jnp.concatenate([_xyz, jnp.zeros((pad, 2), jnp.float32)]).T
    chol_t = jnp.concatenate([_cholesky, jnp.ones((pad, 3), jnp.float32)]).T
    fdc_t = jnp.concatenate([_features_dc, jnp.zeros((pad, 3), jnp.float32)]).T
    op_t = jnp.concatenate([_opacity, jnp.zeros((pad, 1), jnp.float32)]).T

    pt = _project(xyz_t, chol_t, fdc_t, op_t)

    dummy = jnp.zeros((8, PW), jnp.float32)
    dummy = dummy.at[:, 0].set(1e9).at[:, 1].set(1e9)
    dummy = dummy.at[:, 2].set(0.5).at[:, 4].set(0.5)
    tbl = jnp.concatenate([pt.T, dummy], axis=0)

    tp, cnts = _make_bin_kernel()(pt, tbl)
    out = _raster(tp, cnts)

    img = out.reshape(TGY, TGX, 3, TS, TS)
    img = img.transpose(2, 0, 3, 1, 4).reshape(1, 3, H, W)
    return img


# speedup vs baseline: 2.0668x; 2.0668x over previous
import functools

import jax
import jax.numpy as jnp
from jax import lax
from jax.experimental import pallas as pl
from jax.experimental.pallas import tpu as pltpu
from jax.experimental.pallas import tpu_sc as plsc

N = 20000
H = 256
W = 256
GP = 20480
GCH = GP // 16
TS = 16
TGX = W // TS
TGY = H // TS
T = TGX * TGY
K = 448
PW = 16
TWO_T = 24.0
RCH = 64
NKC = K // RCH
NSUB = 32
TPS = T // NSUB
PBLK = 2048


def _proj_kernel(xyz_ref, chol_ref, fdc_ref, op_ref, out_ref):
    i = pl.program_id(0)
    mx = jnp.tanh(xyz_ref[0:1, :])
    my = jnp.tanh(xyz_ref[1:2, :])
    x = 0.5 * (mx + 1.0) * float(W)
    y = 0.5 * (my + 1.0) * float(H)
    l1 = chol_ref[0:1, :] + 0.5
    l2 = chol_ref[1:2, :]
    l3 = chol_ref[2:3, :] + 0.5
    a = l1 * l1
    b = l1 * l2
    c = l2 * l2 + l3 * l3
    inv_det = 1.0 / (a * c - b * b)
    e = 0.5 * c * inv_det
    f = -b * inv_det
    g = 0.5 * a * inv_det
    valid = (i * PBLK + lax.broadcasted_iota(jnp.int32, (1, PBLK), 1)) < N
    r2 = jnp.where(valid, TWO_T * (a + c), -1.0)
    cols = fdc_ref[...] * op_ref[0:1, :]
    zero = jnp.zeros((1, PBLK), jnp.float32)
    out_ref[...] = jnp.concatenate(
        [x, y, e, f, g, cols[0:1], cols[1:2], cols[2:3], r2,
         zero, zero, zero, zero, zero, zero, zero], axis=0)


def _project(xyz_t, chol_t, fdc_t, op_t):
    return pl.pallas_call(
        _proj_kernel,
        grid=(GP // PBLK,),
        in_specs=[
            pl.BlockSpec((2, PBLK), lambda i: (0, i)),
            pl.BlockSpec((3, PBLK), lambda i: (0, i)),
            pl.BlockSpec((3, PBLK), lambda i: (0, i)),
            pl.BlockSpec((1, PBLK), lambda i: (0, i)),
        ],
        out_specs=pl.BlockSpec((PW, PBLK), lambda i: (0, i)),
        out_shape=jax.ShapeDtypeStruct((PW, GP), jnp.float32),
    )(xyz_t, chol_t, fdc_t, op_t)


@functools.cache
def _make_bin_kernel():
    mesh = plsc.VectorSubcoreMesh(core_axis_name="c", subcore_axis_name="s")
    return functools.partial(
        pl.kernel,
        mesh=mesh,
        compiler_params=pltpu.CompilerParams(
            use_tc_tiling_on_sc=False, needs_layout_passes=False),
        out_type=[
            jax.ShapeDtypeStruct((T, K, PW), jnp.float32),
            jax.ShapeDtypeStruct((T,), jnp.int32),
        ],
        scratch_types=[
            pltpu.VMEM((GP,), jnp.float32),
            pltpu.VMEM((GP,), jnp.float32),
            pltpu.VMEM((GP,), jnp.float32),
            pltpu.VMEM((2048,), jnp.int32),
            pltpu.VMEM((K, PW), jnp.float32),
            pltpu.VMEM((16,), jnp.int32),
            pltpu.SemaphoreType.DMA,
        ],
    )(_bin_kernel)


def _bin_kernel(pt_hbm, tbl_hbm, tp_hbm, cnt_hbm, xs, ys, r2s, ids, rows,
                cvec, sem):
    c_ = lax.axis_index("c")
    s_ = lax.axis_index("s")
    wid = s_ * 2 + c_
    pltpu.sync_copy(pt_hbm.at[0], xs)
    pltpu.sync_copy(pt_hbm.at[1], ys)
    pltpu.sync_copy(pt_hbm.at[8], r2s)

    cvals = jnp.zeros((16,), jnp.int32)
    for k in range(TPS):
        tile = wid * TPS + k
        ty = tile // TGX
        tx = tile % TGX
        x0 = tx.astype(jnp.float32) * float(TS) + 0.5
        x1 = x0 + float(TS - 1)
        y0 = ty.astype(jnp.float32) * float(TS) + 0.5
        y1 = y0 + float(TS - 1)

        def init_body(j, _):
            ids[pl.ds(j * 16, 16)] = jnp.full((16,), GP, jnp.int32)
            return 0

        lax.fori_loop(0, K // 16, init_body, 0)

        def scan_body(ci, cnt):
            xv = xs[pl.ds(ci * 16, 16)]
            yv = ys[pl.ds(ci * 16, 16)]
            rv = r2s[pl.ds(ci * 16, 16)]
            dx = jnp.maximum(jnp.maximum(x0 - xv, xv - x1), 0.0)
            dy = jnp.maximum(jnp.maximum(y0 - yv, yv - y1), 0.0)
            m = (dx * dx + dy * dy) <= rv
            iv = ci * 16 + lax.iota(jnp.int32, 16)
            mi = m.astype(jnp.int32)
            csum = plsc.cumsum(mi)
            pos = cnt + csum - 1
            plsc.store_scatter(ids, [pos], iv, mask=m)
            return cnt + jnp.sum(mi)

        cnt = lax.fori_loop(0, GCH, scan_body, 0)
        cvals = jnp.where(lax.iota(jnp.int32, 16) == k, cnt, cvals)

        off = 0
        while off < K:
            sz = min(128, K - off)
            pltpu.async_copy(
                tbl_hbm.at[ids.at[pl.ds(off, sz)]],
                rows.at[pl.ds(off, sz)], sem).wait()
            off += sz
        pltpu.sync_copy(rows, tp_hbm.at[tile])

    cvec[...] = cvals
    pltpu.sync_copy(cvec.at[pl.ds(0, TPS)],
                    cnt_hbm.at[pl.ds(wid * TPS, TPS)])


def _raster_kernel(tp_ref, out_ref):
    t = pl.program_id(0)
    ty = t // TGX
    tx = t % TGX
    p = tp_ref[0]
    x = p[:, 0:1]
    y = p[:, 1:2]
    e = p[:, 2:3]
    f = p[:, 3:4]
    g = p[:, 4:5]
    li = lax.broadcasted_iota(jnp.int32, (K, TS * TS), 1)
    pxv = (tx * TS + (li & (TS - 1))).astype(jnp.float32) + 0.5
    pyv = (ty * TS + (li >> 4)).astype(jnp.float32) + 0.5
    dx = pxv - x
    dy = pyv - y
    sig = dx * (e * dx + f * dy) + g * dy * dy
    alpha = jnp.exp(-sig)
    colsT = p[:, 5:8].T
    acc = lax.dot_general(colsT, alpha, (((1,), (0,)), ((), ())),
                          preferred_element_type=jnp.float32)
    out_ref[0] = jnp.clip(acc, 0.0, 1.0)


def _raster(tp, cnts):
    del cnts
    return pl.pallas_call(
        _raster_kernel,
        grid=(T,),
        in_specs=[pl.BlockSpec((1, K, PW), lambda t: (t, 0, 0))],
        out_specs=pl.BlockSpec((1, 3, TS * TS), lambda t: (t, 0, 0)),
        out_shape=jax.ShapeDtypeStruct((T, 3, TS * TS), jnp.float32),
    )(tp)


def kernel(_xyz, _cholesky, _opacity, _features_dc, background):
    pad = GP - N
    xyz_t = jnp.concatenate([_xyz, jnp.zeros((pad, 2), jnp.float32)]).T
    chol_t = jnp.concatenate([_cholesky, jnp.ones((pad, 3), jnp.float32)]).T
    fdc_t = jnp.concatenate([_features_dc, jnp.zeros((pad, 3), jnp.float32)]).T
    op_t = jnp.concatenate([_opacity, jnp.zeros((pad, 1), jnp.float32)]).T

    pt = _project(xyz_t, chol_t, fdc_t, op_t)

    dummy = jnp.zeros((8, PW), jnp.float32)
    dummy = dummy.at[:, 0].set(1e9).at[:, 1].set(1e9)
    dummy = dummy.at[:, 2].set(0.5).at[:, 4].set(0.5)
    tbl = jnp.concatenate([pt.T, dummy], axis=0)

    tp, cnts = _make_bin_kernel()(pt, tbl)
    out = _raster(tp, cnts)

    img = out.reshape(TGY, TGX, 3, TS, TS)
    img = img.transpose(2, 0, 3, 1, 4).reshape(1, 3, H, W)
    return img


# speedup vs baseline: 2.1044x; 1.0182x over previous
import functools

import jax
import jax.numpy as jnp
from jax import lax
from jax.experimental import pallas as pl
from jax.experimental.pallas import tpu as pltpu
from jax.experimental.pallas import tpu_sc as plsc

N = 20000
H = 256
W = 256
GP = 20480
GCH = GP // 16
TS = 16
TGX = W // TS
TGY = H // TS
T = TGX * TGY
K = 448
PW = 16
TWO_T = 24.0
CAND = 3104
NSUB = 32
TPS = T // NSUB
PBLK = 2048


def _proj_kernel(xyz_ref, chol_ref, fdc_ref, op_ref, out_ref):
    i = pl.program_id(0)
    mx = jnp.tanh(xyz_ref[0:1, :])
    my = jnp.tanh(xyz_ref[1:2, :])
    x = 0.5 * (mx + 1.0) * float(W)
    y = 0.5 * (my + 1.0) * float(H)
    l1 = chol_ref[0:1, :] + 0.5
    l2 = chol_ref[1:2, :]
    l3 = chol_ref[2:3, :] + 0.5
    a = l1 * l1
    b = l1 * l2
    c = l2 * l2 + l3 * l3
    inv_det = 1.0 / (a * c - b * b)
    e = 0.5 * c * inv_det
    f = -b * inv_det
    g = 0.5 * a * inv_det
    valid = (i * PBLK + lax.broadcasted_iota(jnp.int32, (1, PBLK), 1)) < N
    r2 = jnp.where(valid, TWO_T * (a + c), -1.0)
    cols = fdc_ref[...] * op_ref[0:1, :]
    zero = jnp.zeros((1, PBLK), jnp.float32)
    out_ref[...] = jnp.concatenate(
        [x, y, e, f, g, cols[0:1], cols[1:2], cols[2:3], r2,
         zero, zero, zero, zero, zero, zero, zero], axis=0)


def _project(xyz_t, chol_t, fdc_t, op_t):
    return pl.pallas_call(
        _proj_kernel,
        grid=(GP // PBLK,),
        in_specs=[
            pl.BlockSpec((2, PBLK), lambda i: (0, i)),
            pl.BlockSpec((3, PBLK), lambda i: (0, i)),
            pl.BlockSpec((3, PBLK), lambda i: (0, i)),
            pl.BlockSpec((1, PBLK), lambda i: (0, i)),
        ],
        out_specs=pl.BlockSpec((PW, PBLK), lambda i: (0, i)),
        out_shape=jax.ShapeDtypeStruct((PW, GP), jnp.float32),
    )(xyz_t, chol_t, fdc_t, op_t)


@functools.cache
def _make_bin_kernel():
    mesh = plsc.VectorSubcoreMesh(core_axis_name="c", subcore_axis_name="s")
    return functools.partial(
        pl.kernel,
        mesh=mesh,
        compiler_params=pltpu.CompilerParams(
            use_tc_tiling_on_sc=False, needs_layout_passes=False),
        out_type=[
            jax.ShapeDtypeStruct((T, K, PW), jnp.float32),
            jax.ShapeDtypeStruct((T,), jnp.int32),
        ],
        scratch_types=[
            pltpu.VMEM((GP,), jnp.float32),
            pltpu.VMEM((GP,), jnp.float32),
            pltpu.VMEM((GP,), jnp.float32),
            pltpu.VMEM((CAND,), jnp.int32),
            pltpu.VMEM((CAND,), jnp.float32),
            pltpu.VMEM((CAND,), jnp.float32),
            pltpu.VMEM((2048,), jnp.int32),
            pltpu.VMEM((K, PW), jnp.float32),
            pltpu.VMEM((16,), jnp.int32),
            pltpu.SemaphoreType.DMA,
        ],
    )(_bin_kernel)


def _bin_kernel(pt_hbm, tbl_hbm, tp_hbm, cnt_hbm, xs, ys, r2s,
                cand_id, cand_x, cand_rx, ids, rows, cvec, sem):
    c_ = lax.axis_index("c")
    s_ = lax.axis_index("s")
    wid = s_ * 2 + c_
    row = wid // 2
    half = wid % 2
    pltpu.sync_copy(pt_hbm.at[0], xs)
    pltpu.sync_copy(pt_hbm.at[1], ys)
    pltpu.sync_copy(pt_hbm.at[8], r2s)

    y0 = row.astype(jnp.float32) * float(TS) + 0.5
    y1 = y0 + float(TS - 1)

    def p1(ci, carry):
        cntv, cnt = carry
        yv = ys[pl.ds(ci * 16, 16)]
        rv = r2s[pl.ds(ci * 16, 16)]
        xv = xs[pl.ds(ci * 16, 16)]
        dy = jnp.maximum(jnp.maximum(y0 - yv, yv - y1), 0.0)
        rx = rv - dy * dy
        m = rx >= 0.0
        mi = jnp.where(m, 1, 0)
        csum = plsc.cumsum(mi)
        pos = cntv + csum - 1
        iv = ci * 16 + lax.iota(jnp.int32, 16)
        plsc.store_scatter(cand_id, [pos], iv, mask=m)
        plsc.store_scatter(cand_x, [pos], xv, mask=m)
        plsc.store_scatter(cand_rx, [pos], rx, mask=m)
        totv = plsc.all_reduce_population_count(m)
        return (cntv + totv, cnt + jnp.sum(mi))

    cntv1, cnt1 = lax.fori_loop(0, GCH, p1, (jnp.zeros((16,), jnp.int32), 0))
    plsc.store_scatter(cand_rx, [cntv1 + lax.iota(jnp.int32, 16)],
                       jnp.full((16,), -1.0, jnp.float32))
    nch = (cnt1 + 15) // 16

    cvals = jnp.zeros((16,), jnp.int32)
    for k in range(TPS):
        tx = half * TPS + k
        tile = wid * TPS + k
        x0 = tx.astype(jnp.float32) * float(TS) + 0.5
        x1 = x0 + float(TS - 1)

        def init_body(j, _):
            ids[pl.ds(j * 16, 16)] = jnp.full((16,), GP, jnp.int32)
            return 0

        lax.fori_loop(0, K // 16, init_body, 0)

        def p2(ci, carry):
            cntv, cnt = carry
            xv = cand_x[pl.ds(ci * 16, 16)]
            rxv = cand_rx[pl.ds(ci * 16, 16)]
            dxc = jnp.maximum(jnp.maximum(x0 - xv, xv - x1), 0.0)
            m = (dxc * dxc) <= rxv
            mi = jnp.where(m, 1, 0)
            csum = plsc.cumsum(mi)
            pos = cntv + csum - 1
            idv = cand_id[pl.ds(ci * 16, 16)]
            plsc.store_scatter(ids, [pos], idv, mask=m)
            totv = plsc.all_reduce_population_count(m)
            return (cntv + totv, cnt + jnp.sum(mi))

        _, cnt2 = lax.fori_loop(0, nch, p2, (jnp.zeros((16,), jnp.int32), 0))
        cvals = jnp.where(lax.iota(jnp.int32, 16) == k, cnt2, cvals)

        off = 0
        while off < K:
            sz = min(128, K - off)
            pltpu.async_copy(
                tbl_hbm.at[ids.at[pl.ds(off, sz)]],
                rows.at[pl.ds(off, sz)], sem).wait()
            off += sz
        pltpu.sync_copy(rows, tp_hbm.at[tile])

    cvec[...] = cvals
    pltpu.sync_copy(cvec.at[pl.ds(0, TPS)],
                    cnt_hbm.at[pl.ds(wid * TPS, TPS)])


def _raster_kernel(tp_ref, out_ref):
    t = pl.program_id(0)
    ty = t // TGX
    tx = t % TGX
    p = tp_ref[0]
    x = p[:, 0:1]
    y = p[:, 1:2]
    e = p[:, 2:3]
    f = p[:, 3:4]
    g = p[:, 4:5]
    li = lax.broadcasted_iota(jnp.int32, (K, TS * TS), 1)
    pxv = (tx * TS + (li & (TS - 1))).astype(jnp.float32) + 0.5
    pyv = (ty * TS + (li >> 4)).astype(jnp.float32) + 0.5
    dx = pxv - x
    dy = pyv - y
    sig = dx * (e * dx + f * dy) + g * dy * dy
    alpha = jnp.exp(-sig)
    colsT = p[:, 5:8].T
    acc = lax.dot_general(colsT, alpha, (((1,), (0,)), ((), ())),
                          preferred_element_type=jnp.float32)
    out_ref[0] = jnp.clip(acc, 0.0, 1.0)


def _raster(tp, cnts):
    del cnts
    return pl.pallas_call(
        _raster_kernel,
        grid=(T,),
        in_specs=[pl.BlockSpec((1, K, PW), lambda t: (t, 0, 0))],
        out_specs=pl.BlockSpec((1, 3, TS * TS), lambda t: (t, 0, 0)),
        out_shape=jax.ShapeDtypeStruct((T, 3, TS * TS), jnp.float32),
    )(tp)


def kernel(_xyz, _cholesky, _opacity, _features_dc, background):
    pad = GP - N
    xyz_t = jnp.concatenate([_xyz, jnp.zeros((pad, 2), jnp.float32)]).T
    chol_t = jnp.concatenate([_cholesky, jnp.ones((pad, 3), jnp.float32)]).T
    fdc_t = jnp.concatenate([_features_dc, jnp.zeros((pad, 3), jnp.float32)]).T
    op_t = jnp.concatenate([_opacity, jnp.zeros((pad, 1), jnp.float32)]).T

    pt = _project(xyz_t, chol_t, fdc_t, op_t)

    dummy = jnp.zeros((8, PW), jnp.float32)
    dummy = dummy.at[:, 0].set(1e9).at[:, 1].set(1e9)
    dummy = dummy.at[:, 2].set(0.5).at[:, 4].set(0.5)
    tbl = jnp.concatenate([pt.T, dummy], axis=0)

    tp, cnts = _make_bin_kernel()(pt, tbl)
    out = _raster(tp, cnts)

    img = out.reshape(TGY, TGX, 3, TS, TS)
    img = img.transpose(2, 0, 3, 1, 4).reshape(1, 3, H, W)
    return img
